# baseline (speedup 1.0000x reference)
import jax
import jax.numpy as jnp
from jax.experimental import pallas as pl
from jax.experimental.pallas import tpu as pltpu

_H, _W = 8, 17
_S = _H * _W
_C = 13
_BT = 96
_L = _BT * _S

_OFFS1 = [i * _W + j for i in range(4) for j in range(4)]
_OFFS2 = [i * _W + j for i in range(2) for j in range(2)]


def _dot0(w, x):
    return jax.lax.dot_general(
        w, x, (((0,), (0,)), ((), ())),
        preferred_element_type=jnp.float32)


def _qnet_kernel(xt_ref, w1_ref, b1_ref, w2_ref, b2_ref, w3_ref, b3_ref,
                 fw1_ref, fb1_ref, fw2_ref, fb2_ref, o_ref, fw1bf_ref):
    bf16 = jnp.bfloat16

    @pl.when(pl.program_id(0) == 0)
    def _():
        fw1bf_ref[...] = fw1_ref[...].astype(bf16)

    xv = jnp.concatenate(
        [xt_ref[b] for b in range(_BT)], axis=1).astype(bf16)

    x1 = jnp.concatenate(
        [jnp.roll(xv, -off, axis=1) if off else xv for off in _OFFS1], axis=0)
    a1 = _dot0(w1_ref[0:16 * _C, :].astype(bf16), x1)
    a1 = jnp.maximum(a1 + b1_ref[...].T, 0.0).astype(bf16)

    x2 = jnp.concatenate(
        [jnp.roll(a1, -off, axis=1) if off else a1 for off in _OFFS2], axis=0)
    a2 = _dot0(w2_ref[...].astype(bf16), x2)
    a2 = jnp.maximum(a2 + b2_ref[...].T, 0.0).astype(bf16)

    a3 = _dot0(w3_ref[...].astype(bf16), a2)
    a3 = jnp.maximum(a3 + b3_ref[...].T, 0.0).astype(bf16)

    rows = a3.T.reshape(_BT, _S, 128)
    keep = jnp.concatenate(
        [rows[:, h * _W:h * _W + 13, :] for h in range(4)], axis=1)

    w1b = fw1bf_ref[...]
    acc = jnp.dot(keep[:, 0, :], w1b[0:128, :],
                  preferred_element_type=jnp.float32)
    for p in range(1, 52):
        acc += jnp.dot(keep[:, p, :], w1b[128 * p:128 * (p + 1), :],
                       preferred_element_type=jnp.float32)
    h = jnp.maximum(acc + fb1_ref[...], 0.0).astype(bf16)
    r = jnp.dot(h, fw2_ref[...].astype(bf16),
                preferred_element_type=jnp.float32)
    o_ref[...] = r + fb2_ref[...]


def kernel(state, conv1_w, conv1_b, conv2_w, conv2_b, conv3_w, conv3_b,
           fc1_w, fc1_b, fc2_w, fc2_b):
    n = state.shape[0]
    bf16 = jnp.bfloat16

    xt = state.reshape(n, _C, _S)

    grid1 = n // _BT
    np_ = fc2_w.shape[1]
    out = pl.pallas_call(
        _qnet_kernel,
        out_shape=jax.ShapeDtypeStruct((n, np_), jnp.float32),
        grid=(grid1,),
        in_specs=[
            pl.BlockSpec((_BT, _C, _S), lambda i: (i, 0, 0)),
            pl.BlockSpec(conv1_w.shape, lambda i: (0, 0)),
            pl.BlockSpec((1, 64), lambda i: (0, 0)),
            pl.BlockSpec(conv2_w.shape, lambda i: (0, 0)),
            pl.BlockSpec((1, 128), lambda i: (0, 0)),
            pl.BlockSpec(conv3_w.shape, lambda i: (0, 0)),
            pl.BlockSpec((1, 128), lambda i: (0, 0)),
            pl.BlockSpec(fc1_w.shape, lambda i: (0, 0)),
            pl.BlockSpec((1, 512), lambda i: (0, 0)),
            pl.BlockSpec(fc2_w.shape, lambda i: (0, 0)),
            pl.BlockSpec((1, np_), lambda i: (0, 0)),
        ],
        out_specs=pl.BlockSpec((_BT, np_), lambda i: (i, 0)),
        scratch_shapes=[pltpu.VMEM(fc1_w.shape, bf16)],
        compiler_params=pltpu.CompilerParams(
            dimension_semantics=("arbitrary",)),
    )(xt, conv1_w, conv1_b, conv2_w, conv2_b, conv3_w, conv3_b,
      fc1_w, fc1_b, fc2_w, fc2_b)

    return out[:n, :6]


# speedup vs baseline: 1.1058x; 1.1058x over previous
import jax
import jax.numpy as jnp
from jax.experimental import pallas as pl
from jax.experimental.pallas import tpu as pltpu

_H, _W = 8, 17
_S = _H * _W
_C = 13
_BT = 64
_L = _BT * _S

_OFFS1 = [i * _W + j for i in range(4) for j in range(4)]
_OFFS2 = [i * _W + j for i in range(2) for j in range(2)]


def _dot0(w, x):
    return jax.lax.dot_general(
        w, x, (((0,), (0,)), ((), ())),
        preferred_element_type=jnp.float32)


def _qnet_kernel(xt_ref, w1_ref, b1_ref, w2_ref, b2_ref, w3_ref, b3_ref,
                 fw1_hbm, fb1_ref, fw2_ref, fb2_ref, o_ref,
                 fw1f_ref, fw1bf_ref, dma_sem):
    bf16 = jnp.bfloat16
    first = pl.program_id(0) == 0
    cp = pltpu.make_async_copy(fw1_hbm, fw1f_ref, dma_sem)

    @pl.when(first)
    def _():
        cp.start()

    xv = jnp.concatenate(
        [xt_ref[b] for b in range(_BT)], axis=1).astype(bf16)

    x1 = jnp.concatenate(
        [jnp.roll(xv, -off, axis=1) if off else xv for off in _OFFS1], axis=0)
    a1 = _dot0(w1_ref[0:16 * _C, :].astype(bf16), x1)
    a1 = jnp.maximum(a1 + b1_ref[...].T, 0.0).astype(bf16)

    x2 = jnp.concatenate(
        [jnp.roll(a1, -off, axis=1) if off else a1 for off in _OFFS2], axis=0)
    a2 = _dot0(w2_ref[...].astype(bf16), x2)
    a2 = jnp.maximum(a2 + b2_ref[...].T, 0.0).astype(bf16)

    a3 = _dot0(w3_ref[...].astype(bf16), a2)
    a3 = jnp.maximum(a3 + b3_ref[...].T, 0.0).astype(bf16)

    rows = a3.T.reshape(_BT, _S, 128)
    keep = jnp.concatenate(
        [rows[:, h * _W:h * _W + 13, :] for h in range(4)], axis=1)

    @pl.when(first)
    def _():
        cp.wait()
        fw1bf_ref[...] = fw1f_ref[...].astype(bf16)

    w1b = fw1bf_ref[...]
    acc = jnp.dot(keep[:, 0, :], w1b[0:128, :],
                  preferred_element_type=jnp.float32)
    for p in range(1, 52):
        acc += jnp.dot(keep[:, p, :], w1b[128 * p:128 * (p + 1), :],
                       preferred_element_type=jnp.float32)
    h = jnp.maximum(acc + fb1_ref[...], 0.0).astype(bf16)
    r = jnp.dot(h, fw2_ref[...].astype(bf16),
                preferred_element_type=jnp.float32)
    o_ref[...] = r + fb2_ref[...]


def kernel(state, conv1_w, conv1_b, conv2_w, conv2_b, conv3_w, conv3_b,
           fc1_w, fc1_b, fc2_w, fc2_b):
    n = state.shape[0]
    bf16 = jnp.bfloat16

    xt = state.reshape(n, _C, _S)

    grid1 = n // _BT
    np_ = fc2_w.shape[1]
    out = pl.pallas_call(
        _qnet_kernel,
        out_shape=jax.ShapeDtypeStruct((n, np_), jnp.float32),
        grid=(grid1,),
        in_specs=[
            pl.BlockSpec((_BT, _C, _S), lambda i: (i, 0, 0)),
            pl.BlockSpec(conv1_w.shape, lambda i: (0, 0)),
            pl.BlockSpec((1, 64), lambda i: (0, 0)),
            pl.BlockSpec(conv2_w.shape, lambda i: (0, 0)),
            pl.BlockSpec((1, 128), lambda i: (0, 0)),
            pl.BlockSpec(conv3_w.shape, lambda i: (0, 0)),
            pl.BlockSpec((1, 128), lambda i: (0, 0)),
            pl.BlockSpec(memory_space=pl.ANY),
            pl.BlockSpec((1, 512), lambda i: (0, 0)),
            pl.BlockSpec(fc2_w.shape, lambda i: (0, 0)),
            pl.BlockSpec((1, np_), lambda i: (0, 0)),
        ],
        out_specs=pl.BlockSpec((_BT, np_), lambda i: (i, 0)),
        scratch_shapes=[
            pltpu.VMEM(fc1_w.shape, jnp.float32),
            pltpu.VMEM(fc1_w.shape, bf16),
            pltpu.SemaphoreType.DMA,
        ],
        compiler_params=pltpu.CompilerParams(
            dimension_semantics=("arbitrary",)),
    )(xt, conv1_w, conv1_b, conv2_w, conv2_b, conv3_w, conv3_b,
      fc1_w, fc1_b, fc2_w, fc2_b)

    return out[:n, :6]
